# baseline (speedup 1.0000x reference)
import functools

import jax
import jax.numpy as jnp
from jax.experimental import pallas as pl
from jax.experimental.pallas import tpu as pltpu

_ALPHA = 0.01
_BLK = 512


def _leaky(v):
    return jnp.where(v >= 0, v, _ALPHA * v)


def _hist_s(aT):
    D, blk = aT.shape
    bins = D
    xmin = jnp.min(aT, axis=0, keepdims=True)
    xmax = jnp.max(aT, axis=0, keepdims=True)
    width = xmax - xmin
    safe_w = jnp.where(width > 0, width, 1.0)
    fidx = jnp.floor((aT - xmin) / safe_w * bins)
    fidx = jnp.minimum(jnp.maximum(fidx, 0.0), float(bins - 1))
    fidx16 = fidx.astype(jnp.bfloat16)

    one = jnp.bfloat16(1.0)
    zero = jnp.bfloat16(0.0)
    sub = jax.lax.broadcasted_iota(jnp.int32, (8, blk), 0)
    G = min(128, bins)

    def body(jg, s8):
        j0f = (jg * G).astype(jnp.float32)
        t = s8
        for h in range(0, G, 8):
            T = jnp.zeros((8, blk), jnp.float32)
            for g in range(8):
                m = jnp.where(
                    fidx16 == (j0f + float(h + g)).astype(jnp.bfloat16),
                    one, zero)
                d = m.shape[0]
                while d > 16:
                    d //= 2
                    m = m[:d] + m[d:]
                c = m.astype(jnp.float32)
                c = c[:8] + c[8:]
                c = c + jnp.roll(c, 4, axis=0)
                c = c + jnp.roll(c, 2, axis=0)
                c = c + jnp.roll(c, 1, axis=0)
                T = jnp.where(sub == g, c, T)
            t = t + T * jnp.log(jnp.maximum(T, 1.0))
        return t

    s8 = jax.lax.fori_loop(0, bins // G, body,
                           jnp.zeros((8, blk), jnp.float32))
    return jnp.sum(s8, axis=0, keepdims=True)


def _net_kernel(x_ref, w1_ref, w2_ref, w3_ref, w4_ref, w5_ref, w6_ref,
                b1_ref, b2_ref, b3_ref, b4_ref, b5_ref, b6_ref,
                out_ref, hs_ref):
    x = x_ref[...]
    h = _leaky(jax.lax.dot_general(
        w1_ref[...], x, (((1,), (1,)), ((), ())),
        preferred_element_type=jnp.float32) + b1_ref[...])
    hs_ref[0, 0, :] = _hist_s(h)[0, :]

    for i, (w_ref, b_ref) in enumerate(
            [(w2_ref, b2_ref), (w3_ref, b3_ref), (w4_ref, b4_ref),
             (w5_ref, b5_ref)]):
        h = _leaky(jnp.dot(w_ref[...], h,
                           preferred_element_type=jnp.float32) + b_ref[...])
        hs_ref[0, i + 1, :] = _hist_s(h)[0, :]

    l6 = jnp.dot(w6_ref[...], h, preferred_element_type=jnp.float32) + b6_ref[...]
    l6 = jnp.maximum(l6, 0.0)
    m = jnp.max(l6, axis=0, keepdims=True)
    z = l6 - m
    lse = jnp.log(jnp.sum(jnp.exp(z), axis=0, keepdims=True))
    out_ref[...] = z - lse


@functools.partial(jax.jit, static_argnames=())
def kernel(x, W1, b1, W2, b2, W3, b3, W4, b4, W5, b5, W6, b6):
    B = x.shape[0]
    nb = B // _BLK
    dims = [256, 128, 128, 128, 32, 10]
    bcasts = [jnp.broadcast_to(b[:, None], (d, _BLK))
              for b, d in zip((b1, b2, b3, b4, b5, b6), dims)]

    wspec = [pl.BlockSpec(w.shape, lambda i: (0, 0))
             for w in (W1, W2, W3, W4, W5, W6)]
    bspec = [pl.BlockSpec(b.shape, lambda i: (0, 0)) for b in bcasts]

    outT, hsp = pl.pallas_call(
        _net_kernel,
        grid=(nb,),
        in_specs=[pl.BlockSpec((_BLK, 784), lambda i: (i, 0))] + wspec + bspec,
        out_specs=[
            pl.BlockSpec((10, _BLK), lambda i: (0, i)),
            pl.BlockSpec((1, 8, _BLK), lambda i: (i, 0, 0)),
        ],
        out_shape=[
            jax.ShapeDtypeStruct((10, B), jnp.float32),
            jax.ShapeDtypeStruct((nb, 8, _BLK), jnp.float32),
        ],
        compiler_params=pltpu.CompilerParams(
            dimension_semantics=("parallel",),
        ),
    )(x, W1, W2, W3, W4, W5, W6, *bcasts)

    out = outT.T
    totals = jnp.sum(hsp, axis=(0, 2))
    Hs = []
    for i, d in enumerate([256, 128, 128, 128, 32]):
        df = jnp.float32(d)
        Hs.append(2.0 * jnp.log(df) - totals[i] / (df * jnp.float32(B)))
    return (out, Hs[0], Hs[1], Hs[2], Hs[3], Hs[4])


# speedup vs baseline: 1.0123x; 1.0123x over previous
import functools

import jax
import jax.numpy as jnp
from jax.experimental import pallas as pl
from jax.experimental.pallas import tpu as pltpu

_ALPHA = 0.01
_BLK = 512


def _leaky(v):
    return jnp.where(v >= 0, v, _ALPHA * v)


def _hist_s(aT):
    D, blk = aT.shape
    bins = D
    xmin = jnp.min(aT, axis=0, keepdims=True)
    xmax = jnp.max(aT, axis=0, keepdims=True)
    width = xmax - xmin
    safe_w = jnp.where(width > 0, width, 1.0)
    fidx = jnp.floor((aT - xmin) / safe_w * bins)
    fidx = jnp.minimum(jnp.maximum(fidx, 0.0), float(bins - 1))
    fidx16 = fidx.astype(jnp.bfloat16)

    one = jnp.bfloat16(1.0)
    zero = jnp.bfloat16(0.0)
    sub = jax.lax.broadcasted_iota(jnp.int32, (8, blk), 0)
    G = min(64, bins)

    def body(jg, s8):
        j0f = (jg * G).astype(jnp.float32)
        t = s8
        for h in range(0, G, 8):
            cs = []
            for g in range(8):
                m = jnp.where(
                    fidx16 == (j0f + float(h + g)).astype(jnp.bfloat16),
                    one, zero)
                d = m.shape[0]
                while d > 16:
                    d //= 2
                    m = m[:d] + m[d:]
                c = m.astype(jnp.float32)
                cs.append(jnp.sum(c, axis=0, keepdims=True))
            T = jnp.concatenate(cs, axis=0)
            t = t + T * jnp.log(jnp.maximum(T, 1.0))
        return t

    s8 = jax.lax.fori_loop(0, bins // G, body,
                           jnp.zeros((8, blk), jnp.float32))
    return jnp.sum(s8, axis=0, keepdims=True)


def _net_kernel(x_ref, w1_ref, w2_ref, w3_ref, w4_ref, w5_ref, w6_ref,
                b1_ref, b2_ref, b3_ref, b4_ref, b5_ref, b6_ref,
                out_ref, hs_ref):
    x = x_ref[...]
    h = _leaky(jax.lax.dot_general(
        w1_ref[...], x, (((1,), (1,)), ((), ())),
        preferred_element_type=jnp.float32) + b1_ref[...])
    hs_ref[0, 0, :] = _hist_s(h)[0, :]

    for i, (w_ref, b_ref) in enumerate(
            [(w2_ref, b2_ref), (w3_ref, b3_ref), (w4_ref, b4_ref),
             (w5_ref, b5_ref)]):
        h = _leaky(jnp.dot(w_ref[...], h,
                           preferred_element_type=jnp.float32) + b_ref[...])
        hs_ref[0, i + 1, :] = _hist_s(h)[0, :]

    l6 = jnp.dot(w6_ref[...], h, preferred_element_type=jnp.float32) + b6_ref[...]
    l6 = jnp.maximum(l6, 0.0)
    m = jnp.max(l6, axis=0, keepdims=True)
    z = l6 - m
    lse = jnp.log(jnp.sum(jnp.exp(z), axis=0, keepdims=True))
    out_ref[...] = z - lse


@functools.partial(jax.jit, static_argnames=())
def kernel(x, W1, b1, W2, b2, W3, b3, W4, b4, W5, b5, W6, b6):
    B = x.shape[0]
    nb = B // _BLK
    dims = [256, 128, 128, 128, 32, 10]
    bcasts = [jnp.broadcast_to(b[:, None], (d, _BLK))
              for b, d in zip((b1, b2, b3, b4, b5, b6), dims)]

    wspec = [pl.BlockSpec(w.shape, lambda i: (0, 0))
             for w in (W1, W2, W3, W4, W5, W6)]
    bspec = [pl.BlockSpec(b.shape, lambda i: (0, 0)) for b in bcasts]

    outT, hsp = pl.pallas_call(
        _net_kernel,
        grid=(nb,),
        in_specs=[pl.BlockSpec((_BLK, 784), lambda i: (i, 0))] + wspec + bspec,
        out_specs=[
            pl.BlockSpec((10, _BLK), lambda i: (0, i)),
            pl.BlockSpec((1, 8, _BLK), lambda i: (i, 0, 0)),
        ],
        out_shape=[
            jax.ShapeDtypeStruct((10, B), jnp.float32),
            jax.ShapeDtypeStruct((nb, 8, _BLK), jnp.float32),
        ],
        compiler_params=pltpu.CompilerParams(
            dimension_semantics=("parallel",),
        ),
    )(x, W1, W2, W3, W4, W5, W6, *bcasts)

    out = outT.T
    totals = jnp.sum(hsp, axis=(0, 2))
    Hs = []
    for i, d in enumerate([256, 128, 128, 128, 32]):
        df = jnp.float32(d)
        Hs.append(2.0 * jnp.log(df) - totals[i] / (df * jnp.float32(B)))
    return (out, Hs[0], Hs[1], Hs[2], Hs[3], Hs[4])


# speedup vs baseline: 1.0126x; 1.0003x over previous
import functools

import jax
import jax.numpy as jnp
from jax.experimental import pallas as pl
from jax.experimental.pallas import tpu as pltpu

_ALPHA = 0.01
_BLK = 512


def _leaky(v):
    return jnp.where(v >= 0, v, _ALPHA * v)


def _hist_s(aT):
    D, blk = aT.shape
    bins = D
    xmin = jnp.min(aT, axis=0, keepdims=True)
    xmax = jnp.max(aT, axis=0, keepdims=True)
    width = xmax - xmin
    safe_w = jnp.where(width > 0, width, 1.0)
    fidx = jnp.floor((aT - xmin) / safe_w * bins)
    fidx = jnp.minimum(jnp.maximum(fidx, 0.0), float(bins - 1))
    fidx16 = fidx.astype(jnp.bfloat16)

    one = jnp.bfloat16(1.0)
    zero = jnp.bfloat16(0.0)
    G = min(64, bins)

    def body(jg, s8):
        j0f = (jg * G).astype(jnp.float32)
        t = s8
        for h in range(0, G, 8):
            cs = []
            for g in range(8):
                m = jnp.where(
                    fidx16 == (j0f + float(h + g)).astype(jnp.bfloat16),
                    one, zero)
                d = m.shape[0]
                while d > 16:
                    d //= 2
                    m = m[:d] + m[d:]
                c = m.astype(jnp.float32)
                cs.append(jnp.sum(c, axis=0, keepdims=True))
            T = jnp.concatenate(cs, axis=0)
            t = t + T * jnp.log(jnp.maximum(T, 1.0))
        return t

    s8 = jax.lax.fori_loop(0, bins // G, body,
                           jnp.zeros((8, blk), jnp.float32))
    return jnp.sum(s8, axis=0, keepdims=True)


def _net_kernel(x_ref, w1_ref, w2_ref, w3_ref, w4_ref, w5_ref, w6_ref,
                b1_ref, b2_ref, b3_ref, b4_ref, b5_ref, b6_ref,
                out_ref, hs_ref):
    x = x_ref[...]
    h = _leaky(jax.lax.dot_general(
        w1_ref[...], x, (((1,), (1,)), ((), ())),
        preferred_element_type=jnp.float32) + b1_ref[...])
    hs_ref[0, 0, :] = _hist_s(h)[0, :]

    for i, (w_ref, b_ref) in enumerate(
            [(w2_ref, b2_ref), (w3_ref, b3_ref), (w4_ref, b4_ref),
             (w5_ref, b5_ref)]):
        h = _leaky(jnp.dot(w_ref[...], h,
                           preferred_element_type=jnp.float32) + b_ref[...])
        hs_ref[0, i + 1, :] = _hist_s(h)[0, :]

    l6 = jnp.dot(w6_ref[...], h, preferred_element_type=jnp.float32) + b6_ref[...]
    l6 = jnp.maximum(l6, 0.0)
    m = jnp.max(l6, axis=0, keepdims=True)
    z = l6 - m
    lse = jnp.log(jnp.sum(jnp.exp(z), axis=0, keepdims=True))
    out_ref[...] = z - lse


@functools.partial(jax.jit, static_argnames=())
def kernel(x, W1, b1, W2, b2, W3, b3, W4, b4, W5, b5, W6, b6):
    B = x.shape[0]
    nb = B // _BLK
    dims = [256, 128, 128, 128, 32, 10]
    bcasts = [jnp.broadcast_to(b[:, None], (d, _BLK))
              for b, d in zip((b1, b2, b3, b4, b5, b6), dims)]

    wspec = [pl.BlockSpec(w.shape, lambda i: (0, 0))
             for w in (W1, W2, W3, W4, W5, W6)]
    bspec = [pl.BlockSpec(b.shape, lambda i: (0, 0)) for b in bcasts]

    outT, hsp = pl.pallas_call(
        _net_kernel,
        grid=(nb,),
        in_specs=[pl.BlockSpec((_BLK, 784), lambda i: (i, 0))] + wspec + bspec,
        out_specs=[
            pl.BlockSpec((10, _BLK), lambda i: (0, i)),
            pl.BlockSpec((1, 8, _BLK), lambda i: (i, 0, 0)),
        ],
        out_shape=[
            jax.ShapeDtypeStruct((10, B), jnp.float32),
            jax.ShapeDtypeStruct((nb, 8, _BLK), jnp.float32),
        ],
        compiler_params=pltpu.CompilerParams(
            dimension_semantics=("parallel",),
        ),
    )(x, W1, W2, W3, W4, W5, W6, *bcasts)

    out = outT.T
    totals = jnp.sum(hsp, axis=(0, 2))
    Hs = []
    for i, d in enumerate([256, 128, 128, 128, 32]):
        df = jnp.float32(d)
        Hs.append(2.0 * jnp.log(df) - totals[i] / (df * jnp.float32(B)))
    return (out, Hs[0], Hs[1], Hs[2], Hs[3], Hs[4])
